# Initial kernel scaffold; baseline (speedup 1.0000x reference)
import jax
import jax.numpy as jnp
from jax.experimental import pallas as pl


def kernel(input, emb_weight):
    raise NotImplementedError("write your pallas kernel here")


# speedup vs baseline: 4.0929x; 4.0929x over previous
import functools

import jax
import jax.numpy as jnp
from jax import lax
from jax.experimental import pallas as pl
from jax.experimental.pallas import tpu as pltpu
from jax.experimental.pallas import tpu_sc as plsc

_VOCAB = 100000
_D = 64
_B = 4096
_T = 50
_N = _B * _T
_NW = 32
_PER_W = _N // _NW
_CH = 128
_NCH = _PER_W // _CH


def _emb_body(table_hbm, idx_hbm, out_hbm, idx_v, rows_v, sem):
    wid = lax.axis_index("s") * 2 + lax.axis_index("c")
    base = wid * _PER_W
    pltpu.sync_copy(idx_hbm.at[wid], idx_v)

    def chunk(j, carry):
        pltpu.async_copy(table_hbm.at[idx_v.at[j]], rows_v, sem).wait()
        pltpu.sync_copy(rows_v, out_hbm.at[pl.ds(base + j * _CH, _CH)])
        return carry

    lax.fori_loop(0, _NCH, chunk, 0)


_emb_call = functools.partial(
    pl.kernel,
    mesh=plsc.VectorSubcoreMesh(core_axis_name="c", subcore_axis_name="s"),
    out_type=jax.ShapeDtypeStruct((_N, _D), jnp.float32),
    scratch_types=[
        pltpu.VMEM((_NCH, _CH), jnp.int32),
        pltpu.VMEM((_CH, _D), jnp.float32),
        pltpu.SemaphoreType.DMA,
    ],
    compiler_params=pltpu.CompilerParams(use_tc_tiling_on_sc=False),
)(_emb_body)


@jax.jit
def kernel(input, emb_weight):
    idx = input.reshape(_NW, _NCH, _CH).astype(jnp.int32)
    out = _emb_call(emb_weight, idx)
    return out.reshape(_B, _T, _D)


# speedup vs baseline: 4.6239x; 1.1297x over previous
import functools

import jax
import jax.numpy as jnp
from jax import lax
from jax.experimental import pallas as pl
from jax.experimental.pallas import tpu as pltpu
from jax.experimental.pallas import tpu_sc as plsc

_VOCAB = 100000
_D = 64
_B = 4096
_T = 50
_N = _B * _T
_NW = 32
_PER_W = _N // _NW
_CH = 128
_NCH = _PER_W // _CH
_K = 5
_G = _NCH // _K
_GR = _K * _CH


def _emb_body(table_hbm, idx_hbm, out_hbm, idx_v, rows_v, gsem, ssem):
    wid = lax.axis_index("s") * 2 + lax.axis_index("c")
    base = wid * _PER_W
    pltpu.sync_copy(idx_hbm.at[wid], idx_v)

    def fire_gathers(g, phase):
        for b in range(_K):
            pltpu.async_copy(
                table_hbm.at[idx_v.at[g * _K + b]],
                rows_v.at[phase, pl.ds(b * _CH, _CH)],
                gsem,
            )

    def wait_gathers():
        for _ in range(_K):
            pltpu.make_async_copy(
                table_hbm.at[idx_v.at[0]],
                rows_v.at[0, pl.ds(0, _CH)],
                gsem,
            ).wait()

    def wait_store():
        pltpu.make_async_copy(
            rows_v.at[0],
            out_hbm.at[pl.ds(base, _GR)],
            ssem,
        ).wait()

    fire_gathers(0, 0)

    def body(g, carry):
        phase = lax.rem(g, 2)
        wait_gathers()

        @pl.when(g > 0)
        def _():
            wait_store()

        pltpu.async_copy(
            rows_v.at[phase],
            out_hbm.at[pl.ds(base + g * _GR, _GR)],
            ssem,
        )

        @pl.when(g < _G - 1)
        def _():
            fire_gathers(g + 1, 1 - phase)

        return carry

    lax.fori_loop(0, _G, body, 0)
    wait_store()


_emb_call = functools.partial(
    pl.kernel,
    mesh=plsc.VectorSubcoreMesh(core_axis_name="c", subcore_axis_name="s"),
    out_type=jax.ShapeDtypeStruct((_N, _D), jnp.float32),
    scratch_types=[
        pltpu.VMEM((_NCH, _CH), jnp.int32),
        pltpu.VMEM((2, _GR, _D), jnp.float32),
        pltpu.SemaphoreType.DMA,
        pltpu.SemaphoreType.DMA,
    ],
    compiler_params=pltpu.CompilerParams(use_tc_tiling_on_sc=False),
)(_emb_body)


@jax.jit
def kernel(input, emb_weight):
    idx = input.reshape(_NW, _NCH, _CH).astype(jnp.int32)
    out = _emb_call(emb_weight, idx)
    return out.reshape(_B, _T, _D)
